# baseline (speedup 1.0000x reference)
import jax
import jax.numpy as jnp
from jax import lax
from jax.experimental import pallas as pl
from jax.experimental.pallas import tpu as pltpu
from jax.experimental.pallas import tpu_sc as plsc

_VOCAB = 1000
_DIM = 64
_CTX = 20
_NEG = 5
_NW = 32
_NP = _DIM // 2
_PH = _NP // 2
_PV = 1024
_LANES = 16


def _pack_pairs(Wt):
    b = Wt.astype(jnp.bfloat16).reshape(_NP, 2, -1)
    u = lax.bitcast_convert_type(b, jnp.uint16).astype(jnp.uint32)
    w = u[:, 0, :] | (u[:, 1, :] << jnp.uint32(16))
    return lax.bitcast_convert_type(w, jnp.int32).reshape(-1)


def _fold_pairs(acc_bf32):
    w = plsc.bitcast(acc_bf32, jnp.int32)
    lo = plsc.bitcast(w << 16, jnp.float32)
    hi = plsc.bitcast(w & jnp.int32(-65536), jnp.float32)
    return lo + hi


def _sc_body(ctxw_hbm, embw_hbm, ctxidx_hbm, tgt_hbm, r_hbm, pcum_hbm, out_hbm,
             ctxw_v, embw_v, p_v, idx_v, tgt_v, r_v, sc_v):
    wid = lax.axis_index("s") * 2 + lax.axis_index("c")
    rg = tgt_v.shape[0]
    base = wid * rg
    pltpu.sync_copy(ctxw_hbm, ctxw_v)
    pltpu.sync_copy(embw_hbm, embw_v)
    pltpu.sync_copy(pcum_hbm, p_v)
    pltpu.sync_copy(ctxidx_hbm.at[:, pl.ds(base, rg)], idx_v)
    pltpu.sync_copy(tgt_hbm.at[pl.ds(base, rg)], tgt_v)
    pltpu.sync_copy(r_hbm.at[:, pl.ds(base, rg)], r_v)

    zero_bf = jnp.zeros((2 * _LANES,), jnp.bfloat16)
    kstep = jnp.full((_LANES,), _VOCAB, jnp.int32)

    @plsc.parallel_loop(0, rg // _LANES)
    def blk(i):
        b0 = i * _LANES
        ti = tgt_v[pl.ds(b0, _LANES)]
        rs = [r_v[k, pl.ds(b0, _LANES)] for k in range(_NEG)]
        nis = [jnp.zeros((_LANES,), jnp.int32) for _ in range(_NEG)]
        posp = zero_bf
        negp = [zero_bf for _ in range(_NEG)]
        s = _PV // 2
        for half in range(2):
            cvp = [zero_bf for _ in range(_PH)]
            for c in range(_CTX):
                ci = idx_v[c, pl.ds(b0, _LANES)]
                addr = ci + (half * _PH * _VOCAB) if half else ci
                for p in range(_PH):
                    w = plsc.load_gather(ctxw_v, [addr])
                    if p + 1 < _PH:
                        addr = addr + kstep
                    cvp[p] = cvp[p] + plsc.bitcast(w, jnp.bfloat16)
                if half == 0 and c % 2 == 1:
                    for k in range(_NEG):
                        val = plsc.load_gather(p_v, [nis[k] + (s - 1)])
                        nis[k] = nis[k] + jnp.where(
                            val < rs[k], s, 0).astype(jnp.int32)
                    s //= 2
            toff = ti + (half * _PH * _VOCAB) if half else ti
            noff = [ni + (half * _PH * _VOCAB) if half else ni for ni in nis]
            for p in range(_PH):
                tw = plsc.load_gather(embw_v, [toff])
                posp = posp + cvp[p] * plsc.bitcast(tw, jnp.bfloat16)
                for k in range(_NEG):
                    nw = plsc.load_gather(embw_v, [noff[k]])
                    negp[k] = negp[k] + cvp[p] * plsc.bitcast(nw, jnp.bfloat16)
                if p + 1 < _PH:
                    toff = toff + kstep
                    noff = [x + kstep for x in noff]
        inv_ctx = jnp.float32(1.0) / jnp.float32(_CTX)
        sc_v[0, pl.ds(b0, _LANES)] = _fold_pairs(posp) * inv_ctx
        for k in range(_NEG):
            sc_v[1 + k, pl.ds(b0, _LANES)] = _fold_pairs(negp[k]) * inv_ctx

    pltpu.sync_copy(sc_v, out_hbm.at[:, pl.ds(base, rg)])


_TCB = 512


def _tc_score_body(ctx_ref, tgt_ref, r_ref, pp_ref, cw_ref, ew_ref, o_ref):
    pid = pl.program_id(0)
    iota_v = lax.broadcasted_iota(jnp.int32, (_TCB, _PV), 1)
    inv_ctx = jnp.float32(1.0) / jnp.float32(_CTX)
    ctx = ctx_ref[...]
    counts = jnp.zeros((_TCB, _PV), jnp.float32)
    for c in range(_CTX):
        counts = counts + (ctx[:, c:c + 1] == iota_v).astype(jnp.float32)
    cw = cw_ref[...]
    ew = ew_ref[...]
    cvec = jax.lax.dot_general(
        counts.astype(jnp.bfloat16), cw, (((1,), (0,)), ((), ())),
        preferred_element_type=jnp.float32)
    toh = (tgt_ref[...] == iota_v).astype(jnp.bfloat16)
    tvec = jax.lax.dot_general(
        toh, ew, (((1,), (0,)), ((), ())),
        preferred_element_type=jnp.float32)
    pos = jnp.sum(cvec * tvec, axis=1, keepdims=True) * inv_ctx
    tot = jax.nn.log_sigmoid(pos)
    pp = pp_ref[...]
    for k in range(_NEG):
        rk = r_ref[:, k:k + 1]
        nik = jnp.sum((pp < rk).astype(jnp.int32), axis=1, keepdims=True)
        noh = (nik == iota_v).astype(jnp.bfloat16)
        nvec = jax.lax.dot_general(
            noh, ew, (((1,), (0,)), ((), ())),
            preferred_element_type=jnp.float32)
        negk = jnp.sum(cvec * nvec, axis=1, keepdims=True) * inv_ctx
        tot = tot + jax.nn.log_sigmoid(-negk)
    bsum = jnp.reshape(jnp.sum(tot), (1, 1))
    prev = jnp.where(pid == 0, jnp.zeros((1, 1), jnp.float32), o_ref[...])
    o_ref[...] = prev + bsum


def _tc_fin_body(btot, s_ref, t_ref, o_ref):
    x = s_ref[...]
    pos = x[0:1, :]
    neg = x[1:6, :]
    tot = jax.nn.log_sigmoid(pos) + jnp.sum(
        jax.nn.log_sigmoid(-neg), axis=0, keepdims=True)
    total = jnp.sum(tot) + t_ref[0, 0]
    o_ref[:, :] = jnp.reshape(-total / jnp.float32(btot), (1, 1))


_BT = 4096


def kernel(context, target, emb_W, ctx_W, word_freq):
    import functools
    B = context.shape[0]
    bt = _BT if B > _BT else 0
    rg = (B - bt) // _NW
    context = context.astype(jnp.int32)
    target = target.astype(jnp.int32)
    probs = jnp.power(word_freq, 0.75)
    probs = probs / probs.sum()
    p_cuml = jnp.cumsum(probs)
    omuT = jax.device_get(
        1.0 - jax.random.uniform(jax.random.key(1), (B, _NEG),
                                 dtype=jnp.float32)).T
    rT = p_cuml[-1] * jnp.asarray(omuT)
    p_pad = jnp.concatenate(
        [p_cuml, jnp.full((_PV - _VOCAB,), 2.0, jnp.float32)])

    mesh = plsc.VectorSubcoreMesh(core_axis_name="c", subcore_axis_name="s")
    sc = pl.kernel(
        _sc_body,
        out_type=jax.ShapeDtypeStruct((6, B - bt), jnp.float32),
        mesh=mesh,
        compiler_params=pltpu.CompilerParams(needs_layout_passes=False),
        scratch_types=[
            pltpu.VMEM((_NP * _VOCAB,), jnp.int32),
            pltpu.VMEM((_NP * _VOCAB,), jnp.int32),
            pltpu.VMEM((_PV,), jnp.float32),
            pltpu.VMEM((_CTX, rg), jnp.int32),
            pltpu.VMEM((rg,), jnp.int32),
            pltpu.VMEM((_NEG, rg), jnp.float32),
            pltpu.VMEM((6, rg), jnp.float32),
        ],
    )
    pad_rows = _PV - _VOCAB
    cw_pad = jnp.concatenate(
        [ctx_W.astype(jnp.bfloat16),
         jnp.zeros((pad_rows, _DIM), jnp.bfloat16)])
    ew_pad = jnp.concatenate(
        [emb_W.astype(jnp.bfloat16),
         jnp.zeros((pad_rows, _DIM), jnp.bfloat16)])
    r_tc = p_cuml[-1] * jnp.asarray(omuT.T[:bt].copy())
    tc_sum = pl.pallas_call(
        _tc_score_body,
        grid=(bt // _TCB,),
        in_specs=[
            pl.BlockSpec((_TCB, _CTX), lambda i: (i, 0)),
            pl.BlockSpec((_TCB, 1), lambda i: (i, 0)),
            pl.BlockSpec((_TCB, _NEG), lambda i: (i, 0)),
            pl.BlockSpec((1, _PV), lambda i: (0, 0)),
            pl.BlockSpec((_PV, _DIM), lambda i: (0, 0)),
            pl.BlockSpec((_PV, _DIM), lambda i: (0, 0)),
        ],
        out_specs=pl.BlockSpec((1, 1), lambda i: (0, 0)),
        out_shape=jax.ShapeDtypeStruct((1, 1), jnp.float32),
    )(context[:bt], target[:bt].reshape(bt, 1), r_tc,
      p_pad.reshape(1, _PV), cw_pad, ew_pad)

    scores = sc(_pack_pairs(ctx_W.T), _pack_pairs(emb_W.T),
                context.T[:, bt:], target[bt:], rT[:, bt:], p_pad)

    loss = pl.pallas_call(
        functools.partial(_tc_fin_body, B),
        out_shape=jax.ShapeDtypeStruct((1, 1), jnp.float32),
    )(scores, tc_sum)
    return loss[0, 0]


# speedup vs baseline: 1.2211x; 1.2211x over previous
import jax
import jax.numpy as jnp
from jax import lax
from jax.experimental import pallas as pl
from jax.experimental.pallas import tpu as pltpu
from jax.experimental.pallas import tpu_sc as plsc

_VOCAB = 1000
_DIM = 64
_CTX = 20
_NEG = 5
_NW = 32
_NP = _DIM // 2
_PH = _NP // 2
_PV = 1024
_LANES = 16


def _pack_pairs(Wt):
    b = Wt.astype(jnp.bfloat16).reshape(_NP, 2, -1)
    u = lax.bitcast_convert_type(b, jnp.uint16).astype(jnp.uint32)
    w = u[:, 0, :] | (u[:, 1, :] << jnp.uint32(16))
    return lax.bitcast_convert_type(w, jnp.int32).reshape(-1)


def _fold_pairs(acc_bf32):
    w = plsc.bitcast(acc_bf32, jnp.int32)
    lo = plsc.bitcast(w << 16, jnp.float32)
    hi = plsc.bitcast(w & jnp.int32(-65536), jnp.float32)
    return lo + hi


def _sc_body(ctxw_hbm, embw_hbm, ctxidx_hbm, tgt_hbm, r_hbm, pcum_hbm, out_hbm,
             ctxw_v, embw_v, p_v, idx_v, tgt_v, r_v, sc_v):
    wid = lax.axis_index("s") * 2 + lax.axis_index("c")
    rg = tgt_v.shape[0]
    base = wid * rg
    pltpu.sync_copy(ctxw_hbm, ctxw_v)
    pltpu.sync_copy(embw_hbm, embw_v)
    pltpu.sync_copy(pcum_hbm, p_v)
    pltpu.sync_copy(ctxidx_hbm.at[:, pl.ds(base, rg)], idx_v)
    pltpu.sync_copy(tgt_hbm.at[pl.ds(base, rg)], tgt_v)
    pltpu.sync_copy(r_hbm.at[:, pl.ds(base, rg)], r_v)

    zero_bf = jnp.zeros((2 * _LANES,), jnp.bfloat16)
    kstep = jnp.full((_LANES,), _VOCAB, jnp.int32)

    @plsc.parallel_loop(0, rg // _LANES)
    def blk(i):
        b0 = i * _LANES
        ti = tgt_v[pl.ds(b0, _LANES)]
        rs = [r_v[k, pl.ds(b0, _LANES)] for k in range(_NEG)]
        nis = [jnp.zeros((_LANES,), jnp.int32) for _ in range(_NEG)]
        posp = zero_bf
        negp = [zero_bf for _ in range(_NEG)]
        s = _PV // 2
        for half in range(2):
            cvp = [zero_bf for _ in range(_PH)]
            for c in range(_CTX):
                ci = idx_v[c, pl.ds(b0, _LANES)]
                addr = ci + (half * _PH * _VOCAB) if half else ci
                for p in range(_PH):
                    w = plsc.load_gather(ctxw_v, [addr])
                    if p + 1 < _PH:
                        addr = addr + kstep
                    cvp[p] = cvp[p] + plsc.bitcast(w, jnp.bfloat16)
                if half == 0 and c % 2 == 1:
                    for k in range(_NEG):
                        val = plsc.load_gather(p_v, [nis[k] + (s - 1)])
                        nis[k] = nis[k] + jnp.where(
                            val < rs[k], s, 0).astype(jnp.int32)
                    s //= 2
            toff = ti + (half * _PH * _VOCAB) if half else ti
            noff = [ni + (half * _PH * _VOCAB) if half else ni for ni in nis]
            for p in range(_PH):
                tw = plsc.load_gather(embw_v, [toff])
                posp = posp + cvp[p] * plsc.bitcast(tw, jnp.bfloat16)
                for k in range(_NEG):
                    nw = plsc.load_gather(embw_v, [noff[k]])
                    negp[k] = negp[k] + cvp[p] * plsc.bitcast(nw, jnp.bfloat16)
                if p + 1 < _PH:
                    toff = toff + kstep
                    noff = [x + kstep for x in noff]
        inv_ctx = jnp.float32(1.0) / jnp.float32(_CTX)
        sc_v[0, pl.ds(b0, _LANES)] = _fold_pairs(posp) * inv_ctx
        for k in range(_NEG):
            sc_v[1 + k, pl.ds(b0, _LANES)] = _fold_pairs(negp[k]) * inv_ctx

    pltpu.sync_copy(sc_v, out_hbm.at[:, pl.ds(base, rg)])


def _tc_body(s_ref, o_ref):
    x = s_ref[...]
    pos = x[0:1, :]
    neg = x[1:6, :]
    tot = jax.nn.log_sigmoid(pos) + jnp.sum(
        jax.nn.log_sigmoid(-neg), axis=0, keepdims=True)
    o_ref[:, :] = jnp.reshape(-jnp.mean(tot), (1, 1))


def kernel(context, target, emb_W, ctx_W, word_freq):
    B = context.shape[0]
    rg = B // _NW
    context = context.astype(jnp.int32)
    target = target.astype(jnp.int32)
    probs = jnp.power(word_freq, 0.75)
    probs = probs / probs.sum()
    p_cuml = jnp.cumsum(probs)
    omuT = jax.device_get(
        1.0 - jax.random.uniform(jax.random.key(1), (B, _NEG),
                                 dtype=jnp.float32)).T
    rT = p_cuml[-1] * jnp.asarray(omuT)
    p_pad = jnp.concatenate(
        [p_cuml, jnp.full((_PV - _VOCAB,), 2.0, jnp.float32)])

    mesh = plsc.VectorSubcoreMesh(core_axis_name="c", subcore_axis_name="s")
    sc = pl.kernel(
        _sc_body,
        out_type=jax.ShapeDtypeStruct((6, B), jnp.float32),
        mesh=mesh,
        compiler_params=pltpu.CompilerParams(needs_layout_passes=False),
        scratch_types=[
            pltpu.VMEM((_NP * _VOCAB,), jnp.int32),
            pltpu.VMEM((_NP * _VOCAB,), jnp.int32),
            pltpu.VMEM((_PV,), jnp.float32),
            pltpu.VMEM((_CTX, rg), jnp.int32),
            pltpu.VMEM((rg,), jnp.int32),
            pltpu.VMEM((_NEG, rg), jnp.float32),
            pltpu.VMEM((6, rg), jnp.float32),
        ],
    )
    scores = sc(_pack_pairs(ctx_W.T), _pack_pairs(emb_W.T),
                context.T, target, rT, p_pad)

    loss = pl.pallas_call(
        _tc_body,
        out_shape=jax.ShapeDtypeStruct((1, 1), jnp.float32),
    )(scores)
    return loss[0, 0]
